# baseline (speedup 1.0000x reference)
import functools

import jax
import jax.numpy as jnp
from jax import lax
from jax.experimental import pallas as pl
from jax.experimental.pallas import tpu as pltpu
from jax.experimental.pallas import tpu_sc as plsc

N = 10000
D = 128
E = 320000

NC = 2
NS = 16
NW = NC * NS
CHUNK = 128
NCHUNK = 80
NQ = NCHUNK // 4
EPT = CHUNK * NCHUNK
EPAD = NW * EPT
SINK = N
AROWS = 10112
ZR = AROWS // NS

RB = 1000
DSIZE = 10240
HCAT = 32


def _make_agg(d, with_deg):
    mesh = plsc.VectorSubcoreMesh(core_axis_name="c", subcore_axis_name="s",
                                  num_cores=NC, num_subcores=NS)
    out_type = [jax.ShapeDtypeStruct((NC, AROWS, d), jnp.float32)]
    scratch = [
        pltpu.VMEM((4, CHUNK), jnp.int32),
        pltpu.VMEM((4, CHUNK), jnp.int32),
        pltpu.VMEM((4, CHUNK), jnp.int32),
        pltpu.VMEM((4, CHUNK), jnp.int32),
        pltpu.VMEM((CHUNK, d), jnp.float32),
        pltpu.VMEM((CHUNK, d), jnp.float32),
        pltpu.VMEM_SHARED((AROWS, d), jnp.float32),
        pltpu.SemaphoreType.DMA,
        pltpu.SemaphoreType.DMA,
        pltpu.SemaphoreType.DMA,
        pltpu.SemaphoreType.DMA,
    ]
    if with_deg:
        out_type.append(jax.ShapeDtypeStruct((NC, NS, DSIZE), jnp.float32))
        scratch.append(pltpu.VMEM((DSIZE,), jnp.float32))

    def body(h, srcm, dstm, zrows, *rest):
        if with_deg:
            (acc_out, deg_out, srcA, dstA, srcB, dstB, rowsA, rowsB,
             acc_sh, semIA, semIB, semGA, semGB, deg_v) = rest
        else:
            (acc_out, srcA, dstA, srcB, dstB, rowsA, rowsB,
             acc_sh, semIA, semIB, semGA, semGB) = rest
        c = lax.axis_index("c")
        s = lax.axis_index("s")
        w = c * NS + s

        ones = jnp.full((16,), 1.0, jnp.float32)

        def start_idx(q, sv, dv, sem):
            pltpu.async_copy(srcm.at[w, q], sv, sem)
            pltpu.async_copy(dstm.at[w, q], dv, sem)

        def wait_idx(sv, dv, sem):
            pltpu.make_async_copy(srcm.at[w, 0], sv, sem).wait()
            pltpu.make_async_copy(dstm.at[w, 0], dv, sem).wait()

        def start_g(sv, k, rows, sem):
            pltpu.async_copy(h.at[sv.at[k]], rows, sem)

        def wait_g(rows, sem):
            pltpu.make_async_copy(h.at[pl.ds(0, CHUNK)], rows, sem).wait()

        def scat(rows, dv, k):
            pltpu.sync_copy(rows, acc_sh.at[dv.at[k]], add=True)
            if with_deg:
                for t in range(CHUNK // 16):
                    dvec = dv[k, pl.ds(t * 16, 16)]
                    plsc.addupdate_scatter(deg_v, [dvec], ones)

        pltpu.sync_copy(zrows.at[pl.ds(s * ZR, ZR)], acc_sh.at[pl.ds(s * ZR, ZR)])
        if with_deg:
            def zstep(i, carry):
                deg_v[pl.ds(i * 16, 16)] = jnp.zeros((16,), jnp.float32)
                return carry
            lax.fori_loop(0, DSIZE // 16, zstep, 0)
        plsc.subcore_barrier()

        pltpu.sync_copy(srcm.at[w, 0], srcA)
        pltpu.sync_copy(dstm.at[w, 0], dstA)
        start_idx(1, srcB, dstB, semIB)
        start_g(srcA, 0, rowsA, semGA)

        def half(q, cur, nxt):
            (sC, dC, semC) = cur
            (sN, dN, semN) = nxt
            start_g(sC, 1, rowsB, semGB)
            wait_g(rowsA, semGA)
            scat(rowsA, dC, 0)
            start_g(sC, 2, rowsA, semGA)
            wait_g(rowsB, semGB)
            scat(rowsB, dC, 1)
            start_g(sC, 3, rowsB, semGB)
            wait_g(rowsA, semGA)
            scat(rowsA, dC, 2)
            wait_idx(sN, dN, semN)

            @pl.when(q < NQ - 1)
            def _():
                start_g(sN, 0, rowsA, semGA)

            wait_g(rowsB, semGB)
            scat(rowsB, dC, 3)
            start_idx(q + 2, sC, dC, semC)

        bufA = (srcA, dstA, semIA)
        bufB = (srcB, dstB, semIB)

        def octet(o, carry):
            half(2 * o, bufA, bufB)
            half(2 * o + 1, bufB, bufA)
            return carry

        lax.fori_loop(0, NQ // 2, octet, 0)
        wait_idx(srcB, dstB, semIB)

        plsc.subcore_barrier()
        if with_deg:
            pltpu.sync_copy(deg_v, deg_out.at[c, s])
        pltpu.sync_copy(acc_sh.at[pl.ds(s * ZR, ZR)],
                        acc_out.at[c, pl.ds(s * ZR, ZR)])

    if not with_deg:
        out_type = out_type[0]
    return pl.kernel(body, out_type=out_type, mesh=mesh, scratch_types=scratch,
                     compiler_params=pltpu.CompilerParams(needs_layout_passes=False,
                                                          use_tc_tiling_on_sc=False))


_make_agg = functools.lru_cache(None)(_make_agg)


def _agg_deg(*args):
    return _make_agg(D, True)(*args)


def _agg128(*args):
    return _make_agg(D, False)(*args)


def _agg32(*args):
    return _make_agg(HCAT, False)(*args)


def _layer_body(a0, a1, deg, x, Wl, Wr, b, out):
    rd = 1.0 / jnp.maximum(deg[...], 1.0)
    mean = (a0[...] + a1[...]) * rd
    h = (jnp.dot(mean, Wl[...], preferred_element_type=jnp.float32)
         + jnp.dot(x[...], Wr[...], preferred_element_type=jnp.float32)
         + b[...])
    out[...] = jnp.maximum(h, 0.0)


def _layer3_body(a0, a1, deg, x, Wl, Wr, b, Wlcat, out, outp):
    rd = 1.0 / jnp.maximum(deg[...], 1.0)
    mean = (a0[...] + a1[...]) * rd
    h = (jnp.dot(mean, Wl[...], preferred_element_type=jnp.float32)
         + jnp.dot(x[...], Wr[...], preferred_element_type=jnp.float32)
         + b[...])
    h = jnp.maximum(h, 0.0)
    out[...] = h
    outp[...] = jnp.dot(h, Wlcat[...], preferred_element_type=jnp.float32)


def _heads_body(a0, a1, deg, h3, Wrcat, bcat, out):
    rd = 1.0 / jnp.maximum(deg[...], 1.0)
    meanp = (a0[...] + a1[...]) * rd
    out[...] = (meanp
                + jnp.dot(h3[...], Wrcat[...], preferred_element_type=jnp.float32)
                + bcat[...])


def _row_spec(cols):
    return pl.BlockSpec((RB, cols), lambda i: (i, 0))


def _full_spec(rows, cols):
    return pl.BlockSpec((rows, cols), lambda i: (0, 0))


def _tc_layer(a0, a1, deg, x, Wl, Wr, b):
    return pl.pallas_call(
        _layer_body,
        grid=(N // RB,),
        in_specs=[_row_spec(D), _row_spec(D), _row_spec(1), _row_spec(D),
                  _full_spec(D, D), _full_spec(D, D), _full_spec(1, D)],
        out_specs=_row_spec(D),
        out_shape=jax.ShapeDtypeStruct((N, D), jnp.float32),
    )(a0, a1, deg, x, Wl, Wr, b)


def _tc_layer3(a0, a1, deg, x, Wl, Wr, b, Wlcat):
    return pl.pallas_call(
        _layer3_body,
        grid=(N // RB,),
        in_specs=[_row_spec(D), _row_spec(D), _row_spec(1), _row_spec(D),
                  _full_spec(D, D), _full_spec(D, D), _full_spec(1, D),
                  _full_spec(D, HCAT)],
        out_specs=[_row_spec(D), _row_spec(HCAT)],
        out_shape=[jax.ShapeDtypeStruct((N, D), jnp.float32),
                   jax.ShapeDtypeStruct((N, HCAT), jnp.float32)],
    )(a0, a1, deg, x, Wl, Wr, b, Wlcat)


def _tc_heads(a0, a1, deg, h3, Wrcat, bcat):
    return pl.pallas_call(
        _heads_body,
        grid=(N // RB,),
        in_specs=[_row_spec(HCAT), _row_spec(HCAT), _row_spec(1), _row_spec(D),
                  _full_spec(D, HCAT), _full_spec(1, HCAT)],
        out_specs=_row_spec(HCAT),
        out_shape=jax.ShapeDtypeStruct((N, HCAT), jnp.float32),
    )(a0, a1, deg, h3, Wrcat, bcat)


def _pad_cat(ws):
    cat = jnp.concatenate(ws, axis=1)
    return jnp.pad(cat, ((0, 0), (0, HCAT - cat.shape[1])))


def kernel(x, edge_index, c1_Wl, c1_Wr, c1_b, c2_Wl, c2_Wr, c2_b,
           c3_Wl, c3_Wr, c3_b, ca_Wl, ca_Wr, ca_b, cs_Wl, cs_Wr, cs_b,
           ce_Wl, ce_Wr, ce_b):
    src = edge_index[0].astype(jnp.int32)
    dst = edge_index[1].astype(jnp.int32)
    pad = EPAD - E
    def idx_blocks(v, fill):
        vp = jnp.concatenate([v, jnp.full((pad,), fill, jnp.int32)])
        vp = vp.reshape(NW, NQ, 4, CHUNK)
        return jnp.pad(vp, ((0, 0), (0, 2), (0, 0), (0, 0)))

    srcm = idx_blocks(src, 0)
    dstm = idx_blocks(dst, SINK)
    z128 = jnp.zeros((AROWS, D), jnp.float32)
    z32 = jnp.zeros((AROWS, HCAT), jnp.float32)

    accx, degw = _agg_deg(x, srcm, dstm, z128)
    deg = degw.reshape(NW, DSIZE).sum(axis=0)[:N].reshape(N, 1)

    h1 = _tc_layer(accx[0, :N], accx[1, :N], deg, x, c1_Wl, c1_Wr,
                   c1_b.reshape(1, D))
    acc1 = _agg128(h1, srcm, dstm, z128)
    h2 = _tc_layer(acc1[0, :N], acc1[1, :N], deg, h1, c2_Wl, c2_Wr,
                   c2_b.reshape(1, D))
    acc2 = _agg128(h2, srcm, dstm, z128)

    Wlcat = _pad_cat([ca_Wl, cs_Wl, ce_Wl])
    h3, p3 = _tc_layer3(acc2[0, :N], acc2[1, :N], deg, h2, c3_Wl, c3_Wr,
                        c3_b.reshape(1, D), Wlcat)
    accp = _agg32(p3, srcm, dstm, z32)

    Wrcat = _pad_cat([ca_Wr, cs_Wr, ce_Wr])
    bcat = jnp.concatenate([ca_b, cs_b, ce_b,
                            jnp.zeros((HCAT - 28,), jnp.float32)]).reshape(1, HCAT)
    outh = _tc_heads(accp[0, :N], accp[1, :N], deg, h3, Wrcat, bcat)
    return outh[:, :21], outh[:, 21:23], outh[:, 23:28]


# speedup vs baseline: 1.0796x; 1.0796x over previous
import functools

import jax
import jax.numpy as jnp
from jax import lax
from jax.experimental import pallas as pl
from jax.experimental.pallas import tpu as pltpu
from jax.experimental.pallas import tpu_sc as plsc

N = 10000
D = 128
E = 320000

NC = 2
NS = 16
NW = NC * NS
CHUNK = 128
NCHUNK = 80
NPAIR = NCHUNK // 2
EPT = CHUNK * NCHUNK
EPAD = NW * EPT
SINK = N
AROWS = 10112
ZR = AROWS // NS

RB = 1000
DSIZE = 10240
HCAT = 32


def _make_agg(d, with_deg):
    mesh = plsc.VectorSubcoreMesh(core_axis_name="c", subcore_axis_name="s",
                                  num_cores=NC, num_subcores=NS)
    out_type = [jax.ShapeDtypeStruct((NC, AROWS, d), jnp.float32)]
    if with_deg:
        scratch = [
            pltpu.VMEM((NCHUNK, CHUNK), jnp.int32),
            pltpu.VMEM((NCHUNK, CHUNK), jnp.int32),
            pltpu.VMEM((CHUNK, d), jnp.float32),
            pltpu.VMEM_SHARED((AROWS, d), jnp.float32),
            pltpu.SemaphoreType.DMA,
            pltpu.VMEM((DSIZE,), jnp.float32),
        ]
        out_type.append(jax.ShapeDtypeStruct((NC, NS, DSIZE), jnp.float32))
    else:
        scratch = [
            pltpu.VMEM((NCHUNK, CHUNK), jnp.int32),
            pltpu.VMEM((CHUNK,), jnp.int32),
            pltpu.VMEM((CHUNK,), jnp.int32),
            pltpu.VMEM((CHUNK, d), jnp.float32),
            pltpu.VMEM((CHUNK, d), jnp.float32),
            pltpu.VMEM_SHARED((AROWS, d), jnp.float32),
            pltpu.SemaphoreType.DMA,
            pltpu.SemaphoreType.DMA,
            pltpu.SemaphoreType.DMA,
            pltpu.SemaphoreType.DMA,
        ]

    def deg_body(h, srcm, dstm, zrows, acc_out, deg_out,
                 src_v, dst_v, rows_v, acc_sh, sem, deg_v):
        c = lax.axis_index("c")
        s = lax.axis_index("s")
        w = c * NS + s
        pltpu.sync_copy(srcm.at[w], src_v)
        pltpu.sync_copy(dstm.at[w], dst_v)
        pltpu.sync_copy(zrows.at[pl.ds(s * ZR, ZR)], acc_sh.at[pl.ds(s * ZR, ZR)])

        def zstep(i, carry):
            deg_v[pl.ds(i * 16, 16)] = jnp.zeros((16,), jnp.float32)
            return carry
        lax.fori_loop(0, DSIZE // 16, zstep, 0)
        plsc.subcore_barrier()

        ones = jnp.full((16,), 1.0, jnp.float32)

        def step(j, carry):
            pltpu.async_copy(h.at[src_v.at[j]], rows_v, sem).wait()
            pltpu.sync_copy(rows_v, acc_sh.at[dst_v.at[j]], add=True)
            for t in range(CHUNK // 16):
                dvec = dst_v[j, pl.ds(t * 16, 16)]
                plsc.addupdate_scatter(deg_v, [dvec], ones)
            return carry

        lax.fori_loop(0, NCHUNK, step, 0)
        plsc.subcore_barrier()
        pltpu.sync_copy(deg_v, deg_out.at[c, s])
        pltpu.sync_copy(acc_sh.at[pl.ds(s * ZR, ZR)],
                        acc_out.at[c, pl.ds(s * ZR, ZR)])

    def pipe_body(h, srcm, dstm, zrows, acc_out,
                  src_v, dstA, dstB, rowsA, rowsB, acc_sh,
                  semG, semS, semDA, semDB):
        c = lax.axis_index("c")
        s = lax.axis_index("s")
        w = c * NS + s

        def issue_g(j, rows):
            pltpu.async_copy(h.at[src_v.at[j]], rows, semG)

        def wait_g(rows):
            pltpu.make_async_copy(h.at[pl.ds(0, CHUNK)], rows, semG).wait()

        def issue_s(rows, dv):
            pltpu.async_copy(rows, acc_sh.at[dv], semS, add=True)

        def wait_s(rows):
            pltpu.make_async_copy(rows, acc_sh.at[pl.ds(0, CHUNK)], semS).wait()

        def fetch_d(j, dv, sem):
            pltpu.async_copy(dstm.at[w, j], dv, sem)

        def wait_d(dv, sem):
            pltpu.make_async_copy(dstm.at[w, 0], dv, sem).wait()

        pltpu.sync_copy(zrows.at[pl.ds(s * ZR, ZR)], acc_sh.at[pl.ds(s * ZR, ZR)])
        pltpu.sync_copy(srcm.at[w], src_v)
        plsc.subcore_barrier()

        pltpu.sync_copy(dstm.at[w, 0], dstA)
        issue_g(0, rowsA)

        def pair(p, carry):
            wait_g(rowsA)

            @pl.when(p > 0)
            def _():
                wait_d(dstA, semDA)
            issue_s(rowsA, dstA)

            @pl.when(p > 0)
            def _():
                wait_s(rowsB)
            fetch_d(2 * p + 1, dstB, semDB)
            issue_g(2 * p + 1, rowsB)

            wait_g(rowsB)
            wait_d(dstB, semDB)
            issue_s(rowsB, dstB)
            wait_s(rowsA)

            @pl.when(p < NPAIR - 1)
            def _():
                fetch_d(2 * p + 2, dstA, semDA)
                issue_g(2 * p + 2, rowsA)
            return carry

        lax.fori_loop(0, NPAIR, pair, 0)
        wait_s(rowsB)

        plsc.subcore_barrier()
        pltpu.sync_copy(acc_sh.at[pl.ds(s * ZR, ZR)],
                        acc_out.at[c, pl.ds(s * ZR, ZR)])

    if with_deg:
        body = deg_body
    else:
        body = pipe_body
        out_type = out_type[0]
    return pl.kernel(body, out_type=out_type, mesh=mesh, scratch_types=scratch,
                     compiler_params=pltpu.CompilerParams(needs_layout_passes=False,
                                                          use_tc_tiling_on_sc=False))


_make_agg = functools.lru_cache(None)(_make_agg)


def _agg_deg(*args):
    return _make_agg(D, True)(*args)


def _agg128(*args):
    return _make_agg(D, False)(*args)


def _agg32(*args):
    return _make_agg(HCAT, False)(*args)


def _layer_body(a0, a1, deg, x, Wl, Wr, b, out):
    rd = 1.0 / jnp.maximum(deg[...], 1.0)
    mean = (a0[...] + a1[...]) * rd
    h = (jnp.dot(mean, Wl[...], preferred_element_type=jnp.float32)
         + jnp.dot(x[...], Wr[...], preferred_element_type=jnp.float32)
         + b[...])
    out[...] = jnp.maximum(h, 0.0)


def _layer3_body(a0, a1, deg, x, Wl, Wr, b, Wlcat, out, outp):
    rd = 1.0 / jnp.maximum(deg[...], 1.0)
    mean = (a0[...] + a1[...]) * rd
    h = (jnp.dot(mean, Wl[...], preferred_element_type=jnp.float32)
         + jnp.dot(x[...], Wr[...], preferred_element_type=jnp.float32)
         + b[...])
    h = jnp.maximum(h, 0.0)
    out[...] = h
    outp[...] = jnp.dot(h, Wlcat[...], preferred_element_type=jnp.float32)


def _heads_body(a0, a1, deg, h3, Wrcat, bcat, out):
    rd = 1.0 / jnp.maximum(deg[...], 1.0)
    meanp = (a0[...] + a1[...]) * rd
    out[...] = (meanp
                + jnp.dot(h3[...], Wrcat[...], preferred_element_type=jnp.float32)
                + bcat[...])


def _row_spec(cols):
    return pl.BlockSpec((RB, cols), lambda i: (i, 0))


def _full_spec(rows, cols):
    return pl.BlockSpec((rows, cols), lambda i: (0, 0))


def _tc_layer(a0, a1, deg, x, Wl, Wr, b):
    return pl.pallas_call(
        _layer_body,
        grid=(N // RB,),
        in_specs=[_row_spec(D), _row_spec(D), _row_spec(1), _row_spec(D),
                  _full_spec(D, D), _full_spec(D, D), _full_spec(1, D)],
        out_specs=_row_spec(D),
        out_shape=jax.ShapeDtypeStruct((N, D), jnp.float32),
    )(a0, a1, deg, x, Wl, Wr, b)


def _tc_layer3(a0, a1, deg, x, Wl, Wr, b, Wlcat):
    return pl.pallas_call(
        _layer3_body,
        grid=(N // RB,),
        in_specs=[_row_spec(D), _row_spec(D), _row_spec(1), _row_spec(D),
                  _full_spec(D, D), _full_spec(D, D), _full_spec(1, D),
                  _full_spec(D, HCAT)],
        out_specs=[_row_spec(D), _row_spec(HCAT)],
        out_shape=[jax.ShapeDtypeStruct((N, D), jnp.float32),
                   jax.ShapeDtypeStruct((N, HCAT), jnp.float32)],
    )(a0, a1, deg, x, Wl, Wr, b, Wlcat)


def _tc_heads(a0, a1, deg, h3, Wrcat, bcat):
    return pl.pallas_call(
        _heads_body,
        grid=(N // RB,),
        in_specs=[_row_spec(HCAT), _row_spec(HCAT), _row_spec(1), _row_spec(D),
                  _full_spec(D, HCAT), _full_spec(1, HCAT)],
        out_specs=_row_spec(HCAT),
        out_shape=jax.ShapeDtypeStruct((N, HCAT), jnp.float32),
    )(a0, a1, deg, h3, Wrcat, bcat)


def _pad_cat(ws):
    cat = jnp.concatenate(ws, axis=1)
    return jnp.pad(cat, ((0, 0), (0, HCAT - cat.shape[1])))


def kernel(x, edge_index, c1_Wl, c1_Wr, c1_b, c2_Wl, c2_Wr, c2_b,
           c3_Wl, c3_Wr, c3_b, ca_Wl, ca_Wr, ca_b, cs_Wl, cs_Wr, cs_b,
           ce_Wl, ce_Wr, ce_b):
    src = edge_index[0].astype(jnp.int32)
    dst = edge_index[1].astype(jnp.int32)
    pad = EPAD - E
    srcm = jnp.concatenate([src, jnp.zeros((pad,), jnp.int32)]).reshape(NW, NCHUNK, CHUNK)
    dstm = jnp.concatenate([dst, jnp.full((pad,), SINK, jnp.int32)]).reshape(NW, NCHUNK, CHUNK)
    z128 = jnp.zeros((AROWS, D), jnp.float32)
    z32 = jnp.zeros((AROWS, HCAT), jnp.float32)

    accx, degw = _agg_deg(x, srcm, dstm, z128)
    deg = degw.reshape(NW, DSIZE).sum(axis=0)[:N].reshape(N, 1)

    h1 = _tc_layer(accx[0, :N], accx[1, :N], deg, x, c1_Wl, c1_Wr,
                   c1_b.reshape(1, D))
    acc1 = _agg128(h1, srcm, dstm, z128)
    h2 = _tc_layer(acc1[0, :N], acc1[1, :N], deg, h1, c2_Wl, c2_Wr,
                   c2_b.reshape(1, D))
    acc2 = _agg128(h2, srcm, dstm, z128)

    Wlcat = _pad_cat([ca_Wl, cs_Wl, ce_Wl])
    h3, p3 = _tc_layer3(acc2[0, :N], acc2[1, :N], deg, h2, c3_Wl, c3_Wr,
                        c3_b.reshape(1, D), Wlcat)
    accp = _agg32(p3, srcm, dstm, z32)

    Wrcat = _pad_cat([ca_Wr, cs_Wr, ce_Wr])
    bcat = jnp.concatenate([ca_b, cs_b, ce_b,
                            jnp.zeros((HCAT - 28,), jnp.float32)]).reshape(1, HCAT)
    outh = _tc_heads(accp[0, :N], accp[1, :N], deg, h3, Wrcat, bcat)
    return outh[:, :21], outh[:, 21:23], outh[:, 23:28]
